# baseline (speedup 1.0000x reference)
import functools

import jax
import jax.numpy as jnp
from jax import lax
from jax.experimental import pallas as pl
from jax.experimental.pallas import tpu as pltpu
from jax.experimental.pallas import tpu_sc as plsc

LATENT_DIM = 64
LANES = 16
CHUNK = 256


def _make_gather_kernel(batch, nw, nc):
    b_per_w = batch // nw
    n_chunks = b_per_w // CHUNK
    mesh = plsc.VectorSubcoreMesh(core_axis_name="c", subcore_axis_name="s")

    @functools.partial(
        pl.kernel,
        mesh=mesh,
        out_type=jax.ShapeDtypeStruct((batch, 2 * LATENT_DIM), jnp.float32),
        scratch_types=[
            pltpu.VMEM((b_per_w,), jnp.int32),
            pltpu.VMEM((b_per_w,), jnp.int32),
            pltpu.VMEM((CHUNK, 2 * LATENT_DIM), jnp.float32),
            pltpu.SemaphoreType.DMA,
        ],
        compiler_params=pltpu.CompilerParams(needs_layout_passes=False),
    )
    def kg(bidx_hbm, tab_hbm, stage_hbm, idx_v, st_v, rows_v, sem):
        wid = lax.axis_index("s") * nc + lax.axis_index("c")
        base = wid * b_per_w
        pltpu.sync_copy(bidx_hbm.at[pl.ds(base, b_per_w)], idx_v)

        def st_body(j, carry):
            sl = pl.ds(j * LANES, LANES)
            st_v[sl] = lax.shift_right_logical(idx_v[sl], 1)
            return carry

        lax.fori_loop(0, b_per_w // LANES, st_body, 0)

        def chunk_body(c, carry):
            cbase = c * CHUNK
            pltpu.async_copy(
                tab_hbm.at[st_v.at[pl.ds(cbase, CHUNK)]], rows_v,
                sem).wait()
            pltpu.sync_copy(rows_v,
                            stage_hbm.at[pl.ds(base + cbase, CHUNK), :])
            return carry

        lax.fori_loop(0, n_chunks, chunk_body, 0)

    return kg


def _make_dot_kernel(batch, nw, nc):
    b_per_w = batch // nw
    n_chunks = b_per_w // CHUNK
    mesh = plsc.VectorSubcoreMesh(core_axis_name="c", subcore_axis_name="s")

    @functools.partial(
        pl.kernel,
        mesh=mesh,
        out_type=jax.ShapeDtypeStruct((batch,), jnp.float32),
        scratch_types=[
            pltpu.VMEM((b_per_w,), jnp.int32),
            pltpu.VMEM((b_per_w,), jnp.int32),
            pltpu.VMEM((CHUNK, 2 * LATENT_DIM), jnp.float32),
            pltpu.VMEM((CHUNK, 2 * LATENT_DIM), jnp.float32),
            pltpu.VMEM((b_per_w,), jnp.float32),
            pltpu.SemaphoreType.DMA,
            pltpu.SemaphoreType.DMA,
        ],
        compiler_params=pltpu.CompilerParams(needs_layout_passes=False),
    )
    def kd(users_hbm, items_hbm, ustage_hbm, istage_hbm, out_hbm,
           uidx_v, iidx_v, urows_v, irows_v, out_v, sem_u, sem_i):
        wid = lax.axis_index("s") * nc + lax.axis_index("c")
        base = wid * b_per_w
        pltpu.sync_copy(users_hbm.at[pl.ds(base, b_per_w)], uidx_v)
        pltpu.sync_copy(items_hbm.at[pl.ds(base, b_per_w)], iidx_v)
        lane_ids = lax.iota(jnp.int32, LANES)

        def chunk_body(c, carry):
            cbase = c * CHUNK
            cu = pltpu.async_copy(
                ustage_hbm.at[pl.ds(base + cbase, CHUNK), :], urows_v,
                sem_u)
            ci = pltpu.async_copy(
                istage_hbm.at[pl.ds(base + cbase, CHUNK), :], irows_v,
                sem_i)
            cu.wait()
            ci.wait()

            def group_body(g, carry2):
                rows = g * LANES + lane_ids
                uofs = lax.shift_left(
                    jnp.bitwise_and(uidx_v[pl.ds(cbase + g * LANES, LANES)],
                                    1), 6)
                iofs = lax.shift_left(
                    jnp.bitwise_and(iidx_v[pl.ds(cbase + g * LANES, LANES)],
                                    1), 6)
                acc = jnp.zeros((LANES,), jnp.float32)
                for d in range(LATENT_DIM):
                    uv = plsc.load_gather(urows_v, [rows, uofs + d])
                    iv = plsc.load_gather(irows_v, [rows, iofs + d])
                    acc = acc + uv * iv
                out_v[pl.ds(cbase + g * LANES, LANES)] = (
                    1.0 / (1.0 + jnp.exp(-acc)))
                return carry2

            lax.fori_loop(0, CHUNK // LANES, group_body, 0)
            return carry

        lax.fori_loop(0, n_chunks, chunk_body, 0)
        pltpu.sync_copy(out_v, out_hbm.at[pl.ds(base, b_per_w)])

    return kd


def kernel(users, items, embedding_user, embedding_item):
    info = plsc.get_sparse_core_info()
    nw = info.num_cores * info.num_subcores
    nc = info.num_cores
    batch = users.shape[0]
    kg = _make_gather_kernel(batch, nw, nc)
    kd = _make_dot_kernel(batch, nw, nc)
    ui = users.astype(jnp.int32)
    ii = items.astype(jnp.int32)
    ut2 = embedding_user.reshape(embedding_user.shape[0] // 2,
                                 2 * LATENT_DIM)
    it2 = embedding_item.reshape(embedding_item.shape[0] // 2,
                                 2 * LATENT_DIM)
    u_stage = kg(ui, ut2)
    i_stage = kg(ii, it2)
    return kd(ui, ii, u_stage, i_stage)
